# baseline (speedup 1.0000x reference)
import functools

import jax
import jax.numpy as jnp
from jax import lax
from jax.experimental import pallas as pl
from jax.experimental.pallas import tpu as pltpu
from jax.experimental.pallas import tpu_sc as plsc

N_USERS = 25000
N_ITEMS = 25000
N_NODES = N_USERS + N_ITEMS + 2
EMB = 64
LANES = 16
NC = 2
NS = 16

ECH = 112
BLK = 12
NBLK = 38
RPT = BLK * NBLK
TOT_ROWS = NS * RPT
E_PAD = TOT_ROWS * ECH
N_PAD = 50176
H = N_PAD // 2
NDUMP = 8
TILE_OUT = H // NS
NT_Z = N_PAD // NS
OUT_CH = TILE_OUT // ECH

_MESH = dict(core_axis_name="c", subcore_axis_name="s", num_cores=NC,
             num_subcores=NS)
_SC_PARAMS = pltpu.CompilerParams(use_tc_tiling_on_sc=False,
                                  needs_layout_passes=False)


def _rsqrt16(d):
    i = lax.bitcast_convert_type(d, jnp.int32)
    i = jnp.int32(0x5F3759DF) - (i >> 1)
    y = lax.bitcast_convert_type(i, jnp.float32)
    for _ in range(3):
        y = y * (1.5 - 0.5 * d * y * y)
    return y


def _zero_rows(buf):
    z = jnp.zeros((LANES,), jnp.float32)

    def body(r, _):
        for l in range(EMB // LANES):
            buf[r, pl.ds(l * LANES, LANES)] = z
        return 0

    lax.fori_loop(0, ECH, body, 0)


def _zero_acc(acc, zsrc, s):
    for k in range(OUT_CH):
        pltpu.sync_copy(zsrc, acc.at[pl.ds(s * TILE_OUT + k * ECH, ECH)])


def _copy_out(acc, y_out, s, off):
    for k in range(OUT_CH):
        o = s * TILE_OUT + k * ECH
        pltpu.sync_copy(acc.at[pl.ds(o, ECH)], y_out.at[pl.ds(off + o, ECH)])


def _localize_block(didx, off, dump):
    def row(r, _):
        for l in range(ECH // LANES):
            d = didx[r, pl.ds(l * LANES, LANES)]
            t = d - off
            ok = jnp.logical_and(t >= 0, t < H)
            didx[r, pl.ds(l * LANES, LANES)] = jnp.where(ok, t, dump)
        return 0

    lax.fori_loop(0, BLK, row, 0)


def _stage_block(srcr, dstr, b, base, sbuf, dbuf, si):
    row = base + b * BLK
    pltpu.async_copy(srcr.at[pl.ds(row, BLK)], sbuf, si)
    pltpu.async_copy(dstr.at[pl.ds(row, BLK)], dbuf, si)


def _wait_block(srcr, sbuf, dbuf, si):
    pltpu.make_async_copy(srcr.at[pl.ds(0, BLK)], sbuf, si).wait()
    pltpu.make_async_copy(srcr.at[pl.ds(0, BLK)], dbuf, si).wait()


def _dump_vec(off):
    return H + jnp.bitwise_and(lax.iota(jnp.int32, LANES), NDUMP - 1)


def _deg_body(srcr, dstr, a_out, b_out, c_out,
              sidx, didx, ones, zbuf, dbuf, abuf, bbuf, cbuf, sem,
              deg_o, deg_i):
    c = lax.axis_index("c")
    s = lax.axis_index("s")

    z = jnp.zeros((LANES,), jnp.float32)

    def zb(i, _):
        zbuf[pl.ds(i * LANES, LANES)] = z
        return 0

    lax.fori_loop(0, NT_Z // LANES, zb, 0)
    pltpu.sync_copy(zbuf, deg_o.at[pl.ds(s * NT_Z, NT_Z)])
    pltpu.sync_copy(zbuf, deg_i.at[pl.ds(s * NT_Z, NT_Z)])
    one = jnp.ones((LANES,), jnp.float32)
    for l in range(ECH // LANES):
        ones[pl.ds(l * LANES, LANES)] = one
    plsc.subcore_barrier()

    base = s * RPT
    pltpu.sync_copy(srcr.at[pl.ds(base, RPT)], sidx)
    pltpu.sync_copy(dstr.at[pl.ds(base, RPT)], didx)

    def burst(i, _):
        for u in range(4):
            g = i * 4 + u
            pltpu.async_copy(ones, deg_o.at[sidx.at[g]], sem, add=True)
            pltpu.async_copy(ones, deg_i.at[didx.at[g]], sem, add=True)
        for _u in range(8):
            pltpu.make_async_copy(ones, deg_o.at[sidx.at[0]], sem).wait()
        return 0

    lax.fori_loop(0, RPT // 4, burst, 0)
    plsc.subcore_barrier()

    for k in range(OUT_CH):
        gofs = c * H + s * TILE_OUT + k * ECH
        pltpu.sync_copy(deg_o.at[pl.ds(gofs, ECH)], dbuf)
        for l in range(ECH // LANES):
            d = jnp.maximum(dbuf[pl.ds(l * LANES, LANES)], 1.0)
            y = _rsqrt16(d)
            abuf[pl.ds(l * LANES, LANES)] = y
            cbuf[pl.ds(l * LANES, LANES)] = y * d
        pltpu.sync_copy(abuf, a_out.at[pl.ds(gofs, ECH)])
        pltpu.sync_copy(cbuf, c_out.at[pl.ds(gofs, ECH)])
        pltpu.sync_copy(deg_i.at[pl.ds(gofs, ECH)], dbuf)
        for l in range(ECH // LANES):
            d = jnp.maximum(dbuf[pl.ds(l * LANES, LANES)], 1.0)
            bbuf[pl.ds(l * LANES, LANES)] = _rsqrt16(d)
        pltpu.sync_copy(bbuf, b_out.at[pl.ds(gofs, ECH)])


_deg_call = pl.kernel(
    _deg_body,
    out_type=(jax.ShapeDtypeStruct((N_PAD,), jnp.float32),
              jax.ShapeDtypeStruct((N_PAD,), jnp.float32),
              jax.ShapeDtypeStruct((N_PAD,), jnp.float32)),
    mesh=plsc.VectorSubcoreMesh(**_MESH),
    compiler_params=_SC_PARAMS,
    scratch_types=[
        pltpu.VMEM((RPT, ECH), jnp.int32),
        pltpu.VMEM((RPT, ECH), jnp.int32),
        pltpu.VMEM((ECH,), jnp.float32),
        pltpu.VMEM((NT_Z,), jnp.float32),
        pltpu.VMEM((ECH,), jnp.float32),
        pltpu.VMEM((ECH,), jnp.float32),
        pltpu.VMEM((ECH,), jnp.float32),
        pltpu.VMEM((ECH,), jnp.float32),
        pltpu.SemaphoreType.DMA,
        pltpu.VMEM_SHARED((N_PAD,), jnp.float32),
        pltpu.VMEM_SHARED((N_PAD,), jnp.float32),
    ],
)


def _conv_body(xa, srcr, dstr, y_out,
               sA, dA, sB, dB, rows0, rows1, rows2,
               sg0, sg1, sg2, ss0, ss1, ss2, si, acc):
    c = lax.axis_index("c")
    s = lax.axis_index("s")
    off = c * H
    base = s * RPT
    dump = _dump_vec(off)
    rows = (rows0, rows1, rows2)
    sg = (sg0, sg1, sg2)
    ss = (ss0, ss1, ss2)

    _zero_rows(rows0)
    _zero_acc(acc, rows0, s)

    _stage_block(srcr, dstr, 0, base, sA, dA, si)
    _wait_block(srcr, sA, dA, si)
    _localize_block(dA, off, dump)
    plsc.subcore_barrier()
    pltpu.async_copy(xa.at[sA.at[0]], rows0, sg0)
    pltpu.async_copy(xa.at[sA.at[1]], rows1, sg1)

    def do_block(b, cs, cd, ns, nd):
        more = b + 1 < NBLK

        @pl.when(more)
        def _():
            _stage_block(srcr, dstr, b + 1, base, ns, nd, si)

        for u in range(BLK):
            r = u % 3
            r2 = (u + 2) % 3
            pltpu.make_async_copy(xa.at[cs.at[u]], rows[r], sg[r]).wait()
            pltpu.async_copy(rows[r], acc.at[cd.at[u]], ss[r], add=True)
            if u == 0:
                @pl.when(b > 0)
                def _():
                    pltpu.make_async_copy(rows[r2], acc.at[cd.at[0]],
                                          ss[r2]).wait()
            else:
                pltpu.make_async_copy(rows[r2], acc.at[cd.at[0]], ss[r2]).wait()
            if u + 2 < BLK:
                pltpu.async_copy(xa.at[cs.at[u + 2]], rows[r2], sg[r2])
            else:
                @pl.when(more)
                def _():
                    pltpu.async_copy(xa.at[ns.at[u + 2 - BLK]], rows[r2],
                                     sg[r2])
            if u == 5:
                @pl.when(more)
                def _():
                    _wait_block(srcr, ns, nd, si)
                    _localize_block(nd, off, dump)

    def pair(i, _):
        do_block(2 * i, sA, dA, sB, dB)
        do_block(2 * i + 1, sB, dB, sA, dA)
        return 0

    lax.fori_loop(0, NBLK // 2, pair, 0)
    pltpu.make_async_copy(rows[(RPT - 1) % 3], acc.at[dB.at[0]],
                          ss[(RPT - 1) % 3]).wait()
    plsc.subcore_barrier()
    _copy_out(acc, y_out, s, off)


_conv_call = pl.kernel(
    _conv_body,
    out_type=jax.ShapeDtypeStruct((N_PAD, EMB), jnp.float32),
    mesh=plsc.VectorSubcoreMesh(**_MESH),
    compiler_params=_SC_PARAMS,
    scratch_types=[
        pltpu.VMEM((BLK, ECH), jnp.int32),
        pltpu.VMEM((BLK, ECH), jnp.int32),
        pltpu.VMEM((BLK, ECH), jnp.int32),
        pltpu.VMEM((BLK, ECH), jnp.int32),
        pltpu.VMEM((ECH, EMB), jnp.float32),
        pltpu.VMEM((ECH, EMB), jnp.float32),
        pltpu.VMEM((ECH, EMB), jnp.float32),
        pltpu.SemaphoreType.DMA,
        pltpu.SemaphoreType.DMA,
        pltpu.SemaphoreType.DMA,
        pltpu.SemaphoreType.DMA,
        pltpu.SemaphoreType.DMA,
        pltpu.SemaphoreType.DMA,
        pltpu.SemaphoreType.DMA,
        pltpu.VMEM_SHARED((H + NDUMP, EMB), jnp.float32),
    ],
)


def _tda_body(xa, tgt, csrc, srcr, dstr, y_out,
              sA, dA, sB, dB, rx0, rx1, rows_t, ca0, ca1,
              sx0, sx1, st, ssc0, ssc1, si, acc):
    c = lax.axis_index("c")
    s = lax.axis_index("s")
    off = c * H
    base = s * RPT
    dump = _dump_vec(off)
    rx = (rx0, rx1)
    ca = (ca0, ca1)
    sx = (sx0, sx1)
    ssc = (ssc0, ssc1)

    _zero_rows(rx0)
    _zero_acc(acc, rx0, s)

    _stage_block(srcr, dstr, 0, base, sA, dA, si)
    _wait_block(srcr, sA, dA, si)
    plsc.subcore_barrier()
    pltpu.async_copy(xa.at[sA.at[0]], rx0, sx0)
    pltpu.async_copy(csrc.at[sA.at[0]], ca0, sx0)

    def edges4(rows_x, catt, i, _):
        for j in range(4):
            e = i * 4 + j
            xs = [rows_x[e, pl.ds(l * LANES, LANES)]
                  for l in range(EMB // LANES)]
            ts = [rows_t[e, pl.ds(l * LANES, LANES)]
                  for l in range(EMB // LANES)]
            sv = xs[0] * ts[0]
            for l in range(1, EMB // LANES):
                sv = sv + xs[l] * ts[l]
            cb = plsc.load_gather(catt, [jnp.full((LANES,), e, jnp.int32)])
            z = cb * jnp.sum(sv)
            att = 1.0 / (1.0 + jnp.exp(-z))
            for l in range(EMB // LANES):
                rows_x[e, pl.ds(l * LANES, LANES)] = xs[l] * att
        return 0

    def do_chunk(b, cs, cd, nxt_s, u, more):
        p = u % 2
        q = (u + 1) % 2
        dt = pltpu.async_copy(tgt.at[cd.at[u]], rows_t, st)
        pltpu.make_async_copy(xa.at[cs.at[u]], rx[p], sx[p]).wait()
        pltpu.make_async_copy(csrc.at[cs.at[u]], ca[p], sx[p]).wait()
        if u == 0:
            @pl.when(b > 0)
            def _():
                pltpu.make_async_copy(rx[q], acc.at[cd.at[0]], ssc[q]).wait()
        else:
            pltpu.make_async_copy(rx[q], acc.at[cd.at[0]], ssc[q]).wait()
        if u + 1 < BLK:
            pltpu.async_copy(xa.at[cs.at[u + 1]], rx[q], sx[q])
            pltpu.async_copy(csrc.at[cs.at[u + 1]], ca[q], sx[q])
        else:
            @pl.when(more)
            def _():
                pltpu.async_copy(xa.at[nxt_s.at[0]], rx[q], sx[q])
                pltpu.async_copy(csrc.at[nxt_s.at[0]], ca[q], sx[q])
        dt.wait()
        lax.fori_loop(0, ECH // 4,
                      functools.partial(edges4, rx[p], ca[p]), 0)
        for l in range(ECH // LANES):
            d = cd[u, pl.ds(l * LANES, LANES)]
            t = d - off
            ok = jnp.logical_and(t >= 0, t < H)
            cd[u, pl.ds(l * LANES, LANES)] = jnp.where(ok, t, dump)
        pltpu.async_copy(rx[p], acc.at[cd.at[u]], ssc[p], add=True)

    def do_block(b, cs, cd, ns, nd):
        more = b + 1 < NBLK

        @pl.when(more)
        def _():
            _stage_block(srcr, dstr, b + 1, base, ns, nd, si)

        for u in range(BLK):
            do_chunk(b, cs, cd, ns, u, more)
            if u == 5:
                @pl.when(more)
                def _():
                    _wait_block(srcr, ns, nd, si)

    def pair(i, _):
        do_block(2 * i, sA, dA, sB, dB)
        do_block(2 * i + 1, sB, dB, sA, dA)
        return 0

    lax.fori_loop(0, NBLK // 2, pair, 0)
    pltpu.make_async_copy(rx[(RPT - 1) % 2], acc.at[dB.at[0]],
                          ssc[(RPT - 1) % 2]).wait()
    plsc.subcore_barrier()
    _copy_out(acc, y_out, s, off)


_tda_call = pl.kernel(
    _tda_body,
    out_type=jax.ShapeDtypeStruct((N_PAD, EMB), jnp.float32),
    mesh=plsc.VectorSubcoreMesh(**_MESH),
    compiler_params=_SC_PARAMS,
    scratch_types=[
        pltpu.VMEM((BLK, ECH), jnp.int32),
        pltpu.VMEM((BLK, ECH), jnp.int32),
        pltpu.VMEM((BLK, ECH), jnp.int32),
        pltpu.VMEM((BLK, ECH), jnp.int32),
        pltpu.VMEM((ECH, EMB), jnp.float32),
        pltpu.VMEM((ECH, EMB), jnp.float32),
        pltpu.VMEM((ECH, EMB), jnp.float32),
        pltpu.VMEM((ECH,), jnp.float32),
        pltpu.VMEM((ECH,), jnp.float32),
        pltpu.SemaphoreType.DMA,
        pltpu.SemaphoreType.DMA,
        pltpu.SemaphoreType.DMA,
        pltpu.SemaphoreType.DMA,
        pltpu.SemaphoreType.DMA,
        pltpu.SemaphoreType.DMA,
        pltpu.VMEM_SHARED((H + NDUMP, EMB), jnp.float32),
    ],
)


_TC_R = 1792
_TC_GRID = N_PAD // _TC_R


def _scale_body(x_ref, s_ref, o_ref):
    o_ref[...] = x_ref[...] * s_ref[...]


_scale_call = pl.pallas_call(
    _scale_body,
    grid=(_TC_GRID,),
    in_specs=[pl.BlockSpec((_TC_R, EMB), lambda i: (i, 0)),
              pl.BlockSpec((_TC_R, 1), lambda i: (i, 0))],
    out_specs=pl.BlockSpec((_TC_R, EMB), lambda i: (i, 0)),
    out_shape=jax.ShapeDtypeStruct((N_PAD, EMB), jnp.float32),
)


def _post_body(y_ref, b_ref, s_ref, acc_ref, acco_ref, xa_ref, *, inv_k):
    y = y_ref[...] * b_ref[...]
    n2 = jnp.sum(y * y, axis=1, keepdims=True)
    xn = y * lax.rsqrt(jnp.maximum(n2, 1e-24))
    acco_ref[...] = acc_ref[...] + xn * inv_k
    xa_ref[...] = xn * s_ref[...]


def _make_post(inv_k):
    return pl.pallas_call(
        functools.partial(_post_body, inv_k=inv_k),
        grid=(_TC_GRID,),
        in_specs=[pl.BlockSpec((_TC_R, EMB), lambda i: (i, 0)),
                  pl.BlockSpec((_TC_R, 1), lambda i: (i, 0)),
                  pl.BlockSpec((_TC_R, 1), lambda i: (i, 0)),
                  pl.BlockSpec((_TC_R, EMB), lambda i: (i, 0))],
        out_specs=[pl.BlockSpec((_TC_R, EMB), lambda i: (i, 0)),
                   pl.BlockSpec((_TC_R, EMB), lambda i: (i, 0))],
        out_shape=[jax.ShapeDtypeStruct((N_PAD, EMB), jnp.float32),
                   jax.ShapeDtypeStruct((N_PAD, EMB), jnp.float32)],
    )


_post1 = _make_post(1.0)
_post2 = _make_post(0.5)


def _mga_body(x1, x2, x3, x4, x5, w_ref, b_ref, q_ref, o_ref):
    w = w_ref[...]
    bb = b_ref[...]
    qq = q_ref[...]
    xs = [x1[...], x2[...], x3[...], x4[...], x5[...]]
    sc = []
    for x in xs:
        h = jnp.tanh(jnp.dot(x, w, preferred_element_type=jnp.float32) + bb)
        sc.append(jnp.sum(h * qq, axis=1, keepdims=True))
    m = sc[0]
    for k in range(1, 5):
        m = jnp.maximum(m, sc[k])
    es = [jnp.exp(v - m) for v in sc]
    zsum = es[0]
    for k in range(1, 5):
        zsum = zsum + es[k]
    out = xs[0] * (es[0] / zsum)
    for k in range(1, 5):
        out = out + xs[k] * (es[k] / zsum)
    o_ref[...] = out


_mga_call = pl.pallas_call(
    _mga_body,
    grid=(_TC_GRID,),
    in_specs=[pl.BlockSpec((_TC_R, EMB), lambda i: (i, 0))] * 5 +
             [pl.BlockSpec((EMB, EMB), lambda i: (0, 0)),
              pl.BlockSpec((1, EMB), lambda i: (0, 0)),
              pl.BlockSpec((1, EMB), lambda i: (0, 0))],
    out_specs=pl.BlockSpec((_TC_R, EMB), lambda i: (i, 0)),
    out_shape=jax.ShapeDtypeStruct((N_PAD, EMB), jnp.float32),
)


def _prep_edges(e):
    pad = jnp.full((E_PAD - e.shape[1],), N_NODES, jnp.int32)
    sr = jnp.concatenate([e[0], pad]).reshape(TOT_ROWS, ECH)
    dr = jnp.concatenate([e[1], pad]).reshape(TOT_ROWS, ECH)
    return sr, dr


def _propagate(x0, sr, dr, a, b):
    ac = a.reshape(N_PAD, 1)
    bc = b.reshape(N_PAD, 1)
    xa = _scale_call(x0, ac)
    y = _conv_call(xa, sr, dr)
    acc, xa = _post1(y, bc, ac, x0)
    y = _conv_call(xa, sr, dr)
    acc, _ = _post2(y, bc, ac, acc)
    return acc


def _propagate_tda(x0, sr, dr, a, b, cinv, tgt):
    ac = a.reshape(N_PAD, 1)
    bc = b.reshape(N_PAD, 1)
    xa = _scale_call(x0, ac)
    y = _tda_call(xa, tgt, cinv, sr, dr)
    acc, xa = _post1(y, bc, ac, x0)
    y = _tda_call(xa, tgt, cinv, sr, dr)
    acc, _ = _post2(y, bc, ac, acc)
    return acc


def kernel(user_emb, item_emb, edge_ubg, edge_view, edge_buy, edge_view_tcb,
           edge_buy_tib, mga_W, mga_b, mga_q):
    x0 = jnp.zeros((N_PAD, EMB), jnp.float32)
    x0 = x0.at[:N_NODES].set(jnp.concatenate([user_emb, item_emb], axis=0))

    sr_u, dr_u = _prep_edges(edge_ubg)
    sr_v, dr_v = _prep_edges(edge_view)
    sr_b, dr_b = _prep_edges(edge_buy)
    sr_vt, dr_vt = _prep_edges(edge_view_tcb)
    sr_bt, dr_bt = _prep_edges(edge_buy_tib)

    a_u, b_u, _ = _deg_call(sr_u, dr_u)
    a_v, b_v, _ = _deg_call(sr_v, dr_v)
    a_b, b_b, _ = _deg_call(sr_b, dr_b)
    a_vt, b_vt, c_vt = _deg_call(sr_vt, dr_vt)
    a_bt, b_bt, _ = _deg_call(sr_bt, dr_bt)

    ubg = _propagate(x0, sr_u, dr_u, a_u, b_u)
    view = _propagate(ubg, sr_v, dr_v, a_v, b_v)
    buy = _propagate(ubg, sr_b, dr_b, a_b, b_b)
    buy_tib = _propagate(buy, sr_bt, dr_bt, a_bt, b_bt)
    view_tcb = _propagate_tda(view, sr_vt, dr_vt, a_vt, b_vt, c_vt, buy)

    final = _mga_call(ubg, view, buy, view_tcb, buy_tib,
                      mga_W, mga_b.reshape(1, EMB), mga_q.reshape(1, EMB))
    return final[:N_NODES]


# speedup vs baseline: 1.4243x; 1.4243x over previous
import functools

import jax
import jax.numpy as jnp
from jax import lax
from jax.experimental import pallas as pl
from jax.experimental.pallas import tpu as pltpu
from jax.experimental.pallas import tpu_sc as plsc

N_USERS = 25000
N_ITEMS = 25000
N_NODES = N_USERS + N_ITEMS + 2
EMB = 64
LANES = 16
NC = 2
NS = 16

CH = 128
BLK = 8
NBLK = 49
RPT = BLK * NBLK
TOT_ROWS = NS * RPT
E_PAD = TOT_ROWS * CH
N_PAD = 50176
H = N_PAD // 2
NDUMP = 8
TILE_OUT = H // NS
NT_Z = N_PAD // NS
OUT_CHUNKS = [(k * CH, CH) for k in range(TILE_OUT // CH)] + [
    (TILE_OUT - TILE_OUT % CH, TILE_OUT % CH)]

_MESH = dict(core_axis_name="c", subcore_axis_name="s", num_cores=NC,
             num_subcores=NS)
_SC_PARAMS = pltpu.CompilerParams(use_tc_tiling_on_sc=False,
                                  needs_layout_passes=False)


def _rsqrt16(d):
    i = lax.bitcast_convert_type(d, jnp.int32)
    i = jnp.int32(0x5F3759DF) - (i >> 1)
    y = lax.bitcast_convert_type(i, jnp.float32)
    for _ in range(3):
        y = y * (1.5 - 0.5 * d * y * y)
    return y


def _zero_rows(buf):
    z = jnp.zeros((LANES,), jnp.float32)

    def body(r, _):
        for l in range(EMB // LANES):
            buf[r, pl.ds(l * LANES, LANES)] = z
        return 0

    lax.fori_loop(0, CH, body, 0)


def _zero_acc(acc, zsrc, s):
    for ofs, size in OUT_CHUNKS:
        pltpu.sync_copy(zsrc.at[pl.ds(0, size)],
                        acc.at[pl.ds(s * TILE_OUT + ofs, size)])


def _copy_out(acc, y_out, s, off):
    for ofs, size in OUT_CHUNKS:
        o = s * TILE_OUT + ofs
        pltpu.sync_copy(acc.at[pl.ds(o, size)], y_out.at[pl.ds(off + o, size)])


def _dump_vec():
    return H + jnp.bitwise_and(lax.iota(jnp.int32, LANES), NDUMP - 1)


def _localize_block(didx, off, dump):
    for r in range(BLK):
        for l in range(CH // LANES):
            d = didx[r, pl.ds(l * LANES, LANES)]
            t = d - off
            ok = jnp.logical_and(t >= 0, t < H)
            didx[r, pl.ds(l * LANES, LANES)] = jnp.where(ok, t, dump)


def _stage_block(srcr, dstr, b, base, sbuf, dbuf, si):
    row = base + b * BLK
    pltpu.async_copy(srcr.at[pl.ds(row, BLK)], sbuf, si)
    pltpu.async_copy(dstr.at[pl.ds(row, BLK)], dbuf, si)


def _wait_block(srcr, sbuf, dbuf, si):
    pltpu.make_async_copy(srcr.at[pl.ds(0, BLK)], sbuf, si).wait()
    pltpu.make_async_copy(srcr.at[pl.ds(0, BLK)], dbuf, si).wait()


def _deg_body(srcr, dstr, a_out, b_out, c_out,
              sidx, didx, ones, zbuf, dbuf, abuf, bbuf, cbuf, sem,
              deg_o, deg_i):
    c = lax.axis_index("c")
    s = lax.axis_index("s")

    z = jnp.zeros((LANES,), jnp.float32)

    def zb(i, _):
        zbuf[pl.ds(i * LANES, LANES)] = z
        return 0

    lax.fori_loop(0, NT_Z // LANES, zb, 0)
    pltpu.sync_copy(zbuf, deg_o.at[pl.ds(s * NT_Z, NT_Z)])
    pltpu.sync_copy(zbuf, deg_i.at[pl.ds(s * NT_Z, NT_Z)])
    one = jnp.ones((LANES,), jnp.float32)
    for l in range(CH // LANES):
        ones[pl.ds(l * LANES, LANES)] = one
    plsc.subcore_barrier()

    base = s * RPT
    pltpu.sync_copy(srcr.at[pl.ds(base, RPT)], sidx)
    pltpu.sync_copy(dstr.at[pl.ds(base, RPT)], didx)

    def burst(i, _):
        for u in range(4):
            g = i * 4 + u
            pltpu.async_copy(ones, deg_o.at[sidx.at[g]], sem, add=True)
            pltpu.async_copy(ones, deg_i.at[didx.at[g]], sem, add=True)
        for _u in range(8):
            pltpu.make_async_copy(ones, deg_o.at[sidx.at[0]], sem).wait()
        return 0

    lax.fori_loop(0, RPT // 4, burst, 0)
    plsc.subcore_barrier()

    for ofs, size in OUT_CHUNKS:
        gofs = c * H + s * TILE_OUT + ofs
        pltpu.sync_copy(deg_o.at[pl.ds(gofs, size)], dbuf.at[pl.ds(0, size)])
        for l in range(size // LANES):
            d = jnp.maximum(dbuf[pl.ds(l * LANES, LANES)], 1.0)
            y = _rsqrt16(d)
            abuf[pl.ds(l * LANES, LANES)] = y
            cbuf[pl.ds(l * LANES, LANES)] = y * d
        pltpu.sync_copy(abuf.at[pl.ds(0, size)], a_out.at[pl.ds(gofs, size)])
        pltpu.sync_copy(cbuf.at[pl.ds(0, size)], c_out.at[pl.ds(gofs, size)])
        pltpu.sync_copy(deg_i.at[pl.ds(gofs, size)], dbuf.at[pl.ds(0, size)])
        for l in range(size // LANES):
            d = jnp.maximum(dbuf[pl.ds(l * LANES, LANES)], 1.0)
            bbuf[pl.ds(l * LANES, LANES)] = _rsqrt16(d)
        pltpu.sync_copy(bbuf.at[pl.ds(0, size)], b_out.at[pl.ds(gofs, size)])


_deg_call = pl.kernel(
    _deg_body,
    out_type=(jax.ShapeDtypeStruct((N_PAD,), jnp.float32),
              jax.ShapeDtypeStruct((N_PAD,), jnp.float32),
              jax.ShapeDtypeStruct((N_PAD,), jnp.float32)),
    mesh=plsc.VectorSubcoreMesh(**_MESH),
    compiler_params=_SC_PARAMS,
    scratch_types=[
        pltpu.VMEM((RPT, CH), jnp.int32),
        pltpu.VMEM((RPT, CH), jnp.int32),
        pltpu.VMEM((CH,), jnp.float32),
        pltpu.VMEM((NT_Z,), jnp.float32),
        pltpu.VMEM((CH,), jnp.float32),
        pltpu.VMEM((CH,), jnp.float32),
        pltpu.VMEM((CH,), jnp.float32),
        pltpu.VMEM((CH,), jnp.float32),
        pltpu.SemaphoreType.DMA,
        pltpu.VMEM_SHARED((N_PAD,), jnp.float32),
        pltpu.VMEM_SHARED((N_PAD,), jnp.float32),
    ],
)


def _conv_body(xa, srcr, dstr, y_out,
               sA, dA, sB, dB, rows0, rows1,
               sg0, sg1, ss0, ss1, si, acc):
    c = lax.axis_index("c")
    s = lax.axis_index("s")
    off = c * H
    base = s * RPT
    dump = _dump_vec()

    _zero_rows(rows0)
    _zero_acc(acc, rows0, s)

    _stage_block(srcr, dstr, 0, base, sA, dA, si)
    _wait_block(srcr, sA, dA, si)
    _localize_block(dA, off, dump)
    plsc.subcore_barrier()

    rows = (rows0, rows1)
    sg = (sg0, sg1)
    ss = (ss0, ss1)

    def do_block(b, cs, cd, prefetch, ns, nd):
        @pl.when(prefetch)
        def _():
            _stage_block(srcr, dstr, b + 1, base, ns, nd, si)

        pltpu.async_copy(xa.at[cs.at[0]], rows[0], sg[0])
        for u in range(BLK):
            p = u % 2
            q = (u + 1) % 2
            pltpu.make_async_copy(xa.at[cs.at[u]], rows[p], sg[p]).wait()
            pltpu.async_copy(rows[p], acc.at[cd.at[u]], ss[p], add=True)
            if u + 1 < BLK:
                if u >= 1:
                    pltpu.make_async_copy(rows[q], acc.at[cd.at[u]],
                                          ss[q]).wait()
                pltpu.async_copy(xa.at[cs.at[u + 1]], rows[q], sg[q])
        pltpu.make_async_copy(rows[0], acc.at[cd.at[0]], ss[0]).wait()
        pltpu.make_async_copy(rows[1], acc.at[cd.at[0]], ss[1]).wait()

        @pl.when(prefetch)
        def _():
            _wait_block(srcr, ns, nd, si)
            _localize_block(nd, off, dump)

    def pair(i, _):
        b1 = 2 * i + 1
        do_block(b1 - 1, sA, dA, b1 < NBLK, sB, dB)

        @pl.when(b1 < NBLK)
        def _():
            do_block(b1, sB, dB, b1 + 1 < NBLK, sA, dA)

        return 0

    lax.fori_loop(0, (NBLK + 1) // 2, pair, 0)
    plsc.subcore_barrier()
    _copy_out(acc, y_out, s, off)


_conv_call = pl.kernel(
    _conv_body,
    out_type=jax.ShapeDtypeStruct((N_PAD, EMB), jnp.float32),
    mesh=plsc.VectorSubcoreMesh(**_MESH),
    compiler_params=_SC_PARAMS,
    scratch_types=[
        pltpu.VMEM((BLK, CH), jnp.int32),
        pltpu.VMEM((BLK, CH), jnp.int32),
        pltpu.VMEM((BLK, CH), jnp.int32),
        pltpu.VMEM((BLK, CH), jnp.int32),
        pltpu.VMEM((CH, EMB), jnp.float32),
        pltpu.VMEM((CH, EMB), jnp.float32),
        pltpu.SemaphoreType.DMA,
        pltpu.SemaphoreType.DMA,
        pltpu.SemaphoreType.DMA,
        pltpu.SemaphoreType.DMA,
        pltpu.SemaphoreType.DMA,
        pltpu.VMEM_SHARED((H + NDUMP, EMB), jnp.float32),
    ],
)


def _tda_body(xa, tgt, csrc, srcr, dstr, y_out,
              sA, dA, sB, dB, rx0, rx1, rows_t, ca0, ca1,
              sx0, sx1, st, ssc0, ssc1, si, acc):
    c = lax.axis_index("c")
    s = lax.axis_index("s")
    off = c * H
    base = s * RPT
    dump = _dump_vec()
    rx = (rx0, rx1)
    ca = (ca0, ca1)
    sx = (sx0, sx1)
    ssc = (ssc0, ssc1)

    _zero_rows(rx0)
    _zero_acc(acc, rx0, s)

    _stage_block(srcr, dstr, 0, base, sA, dA, si)
    _wait_block(srcr, sA, dA, si)
    plsc.subcore_barrier()
    pltpu.async_copy(xa.at[sA.at[0]], rx0, sx0)
    pltpu.async_copy(csrc.at[sA.at[0]], ca0, sx0)

    def edges4(rows_x, catt, i, _):
        for j in range(4):
            e = i * 4 + j
            xs = [rows_x[e, pl.ds(l * LANES, LANES)]
                  for l in range(EMB // LANES)]
            ts = [rows_t[e, pl.ds(l * LANES, LANES)]
                  for l in range(EMB // LANES)]
            sv = xs[0] * ts[0]
            for l in range(1, EMB // LANES):
                sv = sv + xs[l] * ts[l]
            cb = plsc.load_gather(catt, [jnp.full((LANES,), e, jnp.int32)])
            z = cb * jnp.sum(sv)
            att = 1.0 / (1.0 + jnp.exp(-z))
            for l in range(EMB // LANES):
                rows_x[e, pl.ds(l * LANES, LANES)] = xs[l] * att
        return 0

    def do_chunk(g, cs, cd, nxt_s, u, more):
        p = u % 2
        q = (u + 1) % 2
        dt = pltpu.async_copy(tgt.at[cd.at[u]], rows_t, st)
        pltpu.make_async_copy(xa.at[cs.at[u]], rx[p], sx[p]).wait()
        pltpu.make_async_copy(csrc.at[cs.at[u]], ca[p], sx[p]).wait()

        @pl.when(g > 0)
        def _():
            pltpu.make_async_copy(rx[q], acc.at[cd.at[0]], ssc[q]).wait()

        if u + 1 < BLK:
            pltpu.async_copy(xa.at[cs.at[u + 1]], rx[q], sx[q])
            pltpu.async_copy(csrc.at[cs.at[u + 1]], ca[q], sx[q])
        else:
            @pl.when(more)
            def _():
                pltpu.async_copy(xa.at[nxt_s.at[0]], rx[q], sx[q])
                pltpu.async_copy(csrc.at[nxt_s.at[0]], ca[q], sx[q])
        dt.wait()
        lax.fori_loop(0, CH // 4, functools.partial(edges4, rx[p], ca[p]), 0)
        for l in range(CH // LANES):
            d = cd[u, pl.ds(l * LANES, LANES)]
            t = d - off
            ok = jnp.logical_and(t >= 0, t < H)
            cd[u, pl.ds(l * LANES, LANES)] = jnp.where(ok, t, dump)
        pltpu.async_copy(rx[p], acc.at[cd.at[u]], ssc[p], add=True)

    def do_block(b, cs, cd, prefetch, ns, nd):
        @pl.when(prefetch)
        def _():
            _stage_block(srcr, dstr, b + 1, base, ns, nd, si)

        for u in range(BLK):
            do_chunk(b * BLK + u, cs, cd, ns, u, prefetch)
            if u == 5:
                @pl.when(prefetch)
                def _():
                    _wait_block(srcr, ns, nd, si)

    def pair(i, _):
        b1 = 2 * i + 1
        do_block(b1 - 1, sA, dA, b1 < NBLK, sB, dB)

        @pl.when(b1 < NBLK)
        def _():
            do_block(b1, sB, dB, b1 + 1 < NBLK, sA, dA)

        return 0

    lax.fori_loop(0, (NBLK + 1) // 2, pair, 0)
    pltpu.make_async_copy(rx[(RPT - 1) % 2], acc.at[dB.at[0]],
                          ssc[(RPT - 1) % 2]).wait()
    plsc.subcore_barrier()
    _copy_out(acc, y_out, s, off)


_tda_call = pl.kernel(
    _tda_body,
    out_type=jax.ShapeDtypeStruct((N_PAD, EMB), jnp.float32),
    mesh=plsc.VectorSubcoreMesh(**_MESH),
    compiler_params=_SC_PARAMS,
    scratch_types=[
        pltpu.VMEM((BLK, CH), jnp.int32),
        pltpu.VMEM((BLK, CH), jnp.int32),
        pltpu.VMEM((BLK, CH), jnp.int32),
        pltpu.VMEM((BLK, CH), jnp.int32),
        pltpu.VMEM((CH, EMB), jnp.float32),
        pltpu.VMEM((CH, EMB), jnp.float32),
        pltpu.VMEM((CH, EMB), jnp.float32),
        pltpu.VMEM((CH,), jnp.float32),
        pltpu.VMEM((CH,), jnp.float32),
        pltpu.SemaphoreType.DMA,
        pltpu.SemaphoreType.DMA,
        pltpu.SemaphoreType.DMA,
        pltpu.SemaphoreType.DMA,
        pltpu.SemaphoreType.DMA,
        pltpu.SemaphoreType.DMA,
        pltpu.VMEM_SHARED((H + NDUMP, EMB), jnp.float32),
    ],
)


_TC_R = 1792
_TC_GRID = N_PAD // _TC_R


def _scale_body(x_ref, s_ref, o_ref):
    o_ref[...] = x_ref[...] * s_ref[...]


_scale_call = pl.pallas_call(
    _scale_body,
    grid=(_TC_GRID,),
    in_specs=[pl.BlockSpec((_TC_R, EMB), lambda i: (i, 0)),
              pl.BlockSpec((_TC_R, 1), lambda i: (i, 0))],
    out_specs=pl.BlockSpec((_TC_R, EMB), lambda i: (i, 0)),
    out_shape=jax.ShapeDtypeStruct((N_PAD, EMB), jnp.float32),
)


def _post_body(y_ref, b_ref, s_ref, acc_ref, acco_ref, xa_ref, *, inv_k):
    y = y_ref[...] * b_ref[...]
    n2 = jnp.sum(y * y, axis=1, keepdims=True)
    xn = y * lax.rsqrt(jnp.maximum(n2, 1e-24))
    acco_ref[...] = acc_ref[...] + xn * inv_k
    xa_ref[...] = xn * s_ref[...]


def _make_post(inv_k):
    return pl.pallas_call(
        functools.partial(_post_body, inv_k=inv_k),
        grid=(_TC_GRID,),
        in_specs=[pl.BlockSpec((_TC_R, EMB), lambda i: (i, 0)),
                  pl.BlockSpec((_TC_R, 1), lambda i: (i, 0)),
                  pl.BlockSpec((_TC_R, 1), lambda i: (i, 0)),
                  pl.BlockSpec((_TC_R, EMB), lambda i: (i, 0))],
        out_specs=[pl.BlockSpec((_TC_R, EMB), lambda i: (i, 0)),
                   pl.BlockSpec((_TC_R, EMB), lambda i: (i, 0))],
        out_shape=[jax.ShapeDtypeStruct((N_PAD, EMB), jnp.float32),
                   jax.ShapeDtypeStruct((N_PAD, EMB), jnp.float32)],
    )


_post1 = _make_post(1.0)
_post2 = _make_post(0.5)


def _mga_body(x1, x2, x3, x4, x5, w_ref, b_ref, q_ref, o_ref):
    w = w_ref[...]
    bb = b_ref[...]
    qq = q_ref[...]
    xs = [x1[...], x2[...], x3[...], x4[...], x5[...]]
    sc = []
    for x in xs:
        h = jnp.tanh(jnp.dot(x, w, preferred_element_type=jnp.float32) + bb)
        sc.append(jnp.sum(h * qq, axis=1, keepdims=True))
    m = sc[0]
    for k in range(1, 5):
        m = jnp.maximum(m, sc[k])
    es = [jnp.exp(v - m) for v in sc]
    zsum = es[0]
    for k in range(1, 5):
        zsum = zsum + es[k]
    out = xs[0] * (es[0] / zsum)
    for k in range(1, 5):
        out = out + xs[k] * (es[k] / zsum)
    o_ref[...] = out


_mga_call = pl.pallas_call(
    _mga_body,
    grid=(_TC_GRID,),
    in_specs=[pl.BlockSpec((_TC_R, EMB), lambda i: (i, 0))] * 5 +
             [pl.BlockSpec((EMB, EMB), lambda i: (0, 0)),
              pl.BlockSpec((1, EMB), lambda i: (0, 0)),
              pl.BlockSpec((1, EMB), lambda i: (0, 0))],
    out_specs=pl.BlockSpec((_TC_R, EMB), lambda i: (i, 0)),
    out_shape=jax.ShapeDtypeStruct((N_PAD, EMB), jnp.float32),
)


def _prep_edges(e):
    pad = jnp.full((E_PAD - e.shape[1],), N_NODES, jnp.int32)
    sr = jnp.concatenate([e[0], pad]).reshape(TOT_ROWS, CH)
    dr = jnp.concatenate([e[1], pad]).reshape(TOT_ROWS, CH)
    return sr, dr


def _propagate(x0, sr, dr, a, b):
    ac = a.reshape(N_PAD, 1)
    bc = b.reshape(N_PAD, 1)
    xa = _scale_call(x0, ac)
    y = _conv_call(xa, sr, dr)
    acc, xa = _post1(y, bc, ac, x0)
    y = _conv_call(xa, sr, dr)
    acc, _ = _post2(y, bc, ac, acc)
    return acc


def _propagate_tda(x0, sr, dr, a, b, cinv, tgt):
    ac = a.reshape(N_PAD, 1)
    bc = b.reshape(N_PAD, 1)
    xa = _scale_call(x0, ac)
    y = _tda_call(xa, tgt, cinv, sr, dr)
    acc, xa = _post1(y, bc, ac, x0)
    y = _tda_call(xa, tgt, cinv, sr, dr)
    acc, _ = _post2(y, bc, ac, acc)
    return acc


def kernel(user_emb, item_emb, edge_ubg, edge_view, edge_buy, edge_view_tcb,
           edge_buy_tib, mga_W, mga_b, mga_q):
    x0 = jnp.zeros((N_PAD, EMB), jnp.float32)
    x0 = x0.at[:N_NODES].set(jnp.concatenate([user_emb, item_emb], axis=0))

    sr_u, dr_u = _prep_edges(edge_ubg)
    sr_v, dr_v = _prep_edges(edge_view)
    sr_b, dr_b = _prep_edges(edge_buy)
    sr_vt, dr_vt = _prep_edges(edge_view_tcb)
    sr_bt, dr_bt = _prep_edges(edge_buy_tib)

    a_u, b_u, _ = _deg_call(sr_u, dr_u)
    a_v, b_v, _ = _deg_call(sr_v, dr_v)
    a_b, b_b, _ = _deg_call(sr_b, dr_b)
    a_vt, b_vt, c_vt = _deg_call(sr_vt, dr_vt)
    a_bt, b_bt, _ = _deg_call(sr_bt, dr_bt)

    ubg = _propagate(x0, sr_u, dr_u, a_u, b_u)
    view = _propagate(ubg, sr_v, dr_v, a_v, b_v)
    buy = _propagate(ubg, sr_b, dr_b, a_b, b_b)
    buy_tib = _propagate(buy, sr_bt, dr_bt, a_bt, b_bt)
    view_tcb = _propagate_tda(view, sr_vt, dr_vt, a_vt, b_vt, c_vt, buy)

    final = _mga_call(ubg, view, buy, view_tcb, buy_tib,
                      mga_W, mga_b.reshape(1, EMB), mga_q.reshape(1, EMB))
    return final[:N_NODES]


# speedup vs baseline: 1.9103x; 1.3413x over previous
import functools

import jax
import jax.numpy as jnp
from jax import lax
from jax.experimental import pallas as pl
from jax.experimental.pallas import tpu as pltpu
from jax.experimental.pallas import tpu_sc as plsc

N_USERS = 25000
N_ITEMS = 25000
N_NODES = N_USERS + N_ITEMS + 2
EMB = 64
LANES = 16
NC = 2
NS = 16

CH = 128
BLK = 8
NBLK = 49
RPT = BLK * NBLK
TOT_ROWS = NS * RPT
E_PAD = TOT_ROWS * CH
N_PAD = 50176
H = N_PAD // 2
NDUMP = 8
TILE_OUT = H // NS
NT_Z = N_PAD // NS
OUT_CHUNKS = [(k * CH, CH) for k in range(TILE_OUT // CH)] + [
    (TILE_OUT - TILE_OUT % CH, TILE_OUT % CH)]

_MESH = dict(core_axis_name="c", subcore_axis_name="s", num_cores=NC,
             num_subcores=NS)
_SC_PARAMS = pltpu.CompilerParams(use_tc_tiling_on_sc=False,
                                  needs_layout_passes=False)


def _rsqrt16(d):
    i = lax.bitcast_convert_type(d, jnp.int32)
    i = jnp.int32(0x5F3759DF) - (i >> 1)
    y = lax.bitcast_convert_type(i, jnp.float32)
    for _ in range(3):
        y = y * (1.5 - 0.5 * d * y * y)
    return y


def _zero_rows(buf):
    z = jnp.zeros((LANES,), jnp.float32)

    def body(r, _):
        for l in range(EMB // LANES):
            buf[r, pl.ds(l * LANES, LANES)] = z
        return 0

    lax.fori_loop(0, CH, body, 0)


def _zero_acc(acc, zsrc, s):
    for ofs, size in OUT_CHUNKS:
        pltpu.sync_copy(zsrc.at[pl.ds(0, size)],
                        acc.at[pl.ds(s * TILE_OUT + ofs, size)])


def _copy_out(acc, y_out, s, off):
    for ofs, size in OUT_CHUNKS:
        o = s * TILE_OUT + ofs
        pltpu.sync_copy(acc.at[pl.ds(o, size)], y_out.at[pl.ds(off + o, size)])


def _dump_vec():
    return H + jnp.bitwise_and(lax.iota(jnp.int32, LANES), NDUMP - 1)


def _localize_block(didx, off, dump):
    for r in range(BLK):
        for l in range(CH // LANES):
            d = didx[r, pl.ds(l * LANES, LANES)]
            t = d - off
            ok = jnp.logical_and(t >= 0, t < H)
            didx[r, pl.ds(l * LANES, LANES)] = jnp.where(ok, t, dump)


def _stage_block(srcr, dstr, b, base, sbuf, dbuf, si):
    row = base + b * BLK
    pltpu.async_copy(srcr.at[pl.ds(row, BLK)], sbuf, si)
    pltpu.async_copy(dstr.at[pl.ds(row, BLK)], dbuf, si)


def _wait_block(srcr, sbuf, dbuf, si):
    pltpu.make_async_copy(srcr.at[pl.ds(0, BLK)], sbuf, si).wait()
    pltpu.make_async_copy(srcr.at[pl.ds(0, BLK)], dbuf, si).wait()


def _deg_body(srcr, dstr, a_out, b_out, c_out,
              sidx, didx, ones, zbuf, dbuf, abuf, bbuf, cbuf, sem,
              deg_o, deg_i):
    c = lax.axis_index("c")
    s = lax.axis_index("s")

    z = jnp.zeros((LANES,), jnp.float32)

    def zb(i, _):
        zbuf[pl.ds(i * LANES, LANES)] = z
        return 0

    lax.fori_loop(0, NT_Z // LANES, zb, 0)
    pltpu.sync_copy(zbuf, deg_o.at[pl.ds(s * NT_Z, NT_Z)])
    pltpu.sync_copy(zbuf, deg_i.at[pl.ds(s * NT_Z, NT_Z)])
    one = jnp.ones((LANES,), jnp.float32)
    for l in range(CH // LANES):
        ones[pl.ds(l * LANES, LANES)] = one
    plsc.subcore_barrier()

    base = s * RPT
    pltpu.sync_copy(srcr.at[pl.ds(base, RPT)], sidx)
    pltpu.sync_copy(dstr.at[pl.ds(base, RPT)], didx)

    def burst(i, _):
        for u in range(4):
            g = i * 4 + u
            pltpu.async_copy(ones, deg_o.at[sidx.at[g]], sem, add=True)
            pltpu.async_copy(ones, deg_i.at[didx.at[g]], sem, add=True)
        for _u in range(8):
            pltpu.make_async_copy(ones, deg_o.at[sidx.at[0]], sem).wait()
        return 0

    lax.fori_loop(0, RPT // 4, burst, 0)
    plsc.subcore_barrier()

    for ofs, size in OUT_CHUNKS:
        gofs = c * H + s * TILE_OUT + ofs
        pltpu.sync_copy(deg_o.at[pl.ds(gofs, size)], dbuf.at[pl.ds(0, size)])
        for l in range(size // LANES):
            d = jnp.maximum(dbuf[pl.ds(l * LANES, LANES)], 1.0)
            y = _rsqrt16(d)
            abuf[pl.ds(l * LANES, LANES)] = y
            cbuf[pl.ds(l * LANES, LANES)] = y * d
        pltpu.sync_copy(abuf.at[pl.ds(0, size)], a_out.at[pl.ds(gofs, size)])
        pltpu.sync_copy(cbuf.at[pl.ds(0, size)], c_out.at[pl.ds(gofs, size)])
        pltpu.sync_copy(deg_i.at[pl.ds(gofs, size)], dbuf.at[pl.ds(0, size)])
        for l in range(size // LANES):
            d = jnp.maximum(dbuf[pl.ds(l * LANES, LANES)], 1.0)
            bbuf[pl.ds(l * LANES, LANES)] = _rsqrt16(d)
        pltpu.sync_copy(bbuf.at[pl.ds(0, size)], b_out.at[pl.ds(gofs, size)])


_deg_call = pl.kernel(
    _deg_body,
    out_type=(jax.ShapeDtypeStruct((N_PAD,), jnp.float32),
              jax.ShapeDtypeStruct((N_PAD,), jnp.float32),
              jax.ShapeDtypeStruct((N_PAD,), jnp.float32)),
    mesh=plsc.VectorSubcoreMesh(**_MESH),
    compiler_params=_SC_PARAMS,
    scratch_types=[
        pltpu.VMEM((RPT, CH), jnp.int32),
        pltpu.VMEM((RPT, CH), jnp.int32),
        pltpu.VMEM((CH,), jnp.float32),
        pltpu.VMEM((NT_Z,), jnp.float32),
        pltpu.VMEM((CH,), jnp.float32),
        pltpu.VMEM((CH,), jnp.float32),
        pltpu.VMEM((CH,), jnp.float32),
        pltpu.VMEM((CH,), jnp.float32),
        pltpu.SemaphoreType.DMA,
        pltpu.VMEM_SHARED((N_PAD,), jnp.float32),
        pltpu.VMEM_SHARED((N_PAD,), jnp.float32),
    ],
)


def _conv_body(xa, srcr, dstr, y_out,
               sA, dA, sB, dB, rows0, rows1,
               sg0, sg1, ss0, ss1, si, acc):
    c = lax.axis_index("c")
    s = lax.axis_index("s")
    off = c * H
    base = s * RPT
    dump = _dump_vec()

    _zero_rows(rows0)
    _zero_acc(acc, rows0, s)

    _stage_block(srcr, dstr, 0, base, sA, dA, si)
    _wait_block(srcr, sA, dA, si)
    _localize_block(dA, off, dump)
    plsc.subcore_barrier()

    rows = (rows0, rows1)
    sg = (sg0, sg1)
    ss = (ss0, ss1)

    def do_block(b, cs, cd, prefetch, ns, nd):
        @pl.when(prefetch)
        def _():
            _stage_block(srcr, dstr, b + 1, base, ns, nd, si)

        pltpu.async_copy(xa.at[cs.at[0]], rows[0], sg[0])
        for u in range(BLK):
            p = u % 2
            q = (u + 1) % 2
            pltpu.make_async_copy(xa.at[cs.at[u]], rows[p], sg[p]).wait()
            pltpu.async_copy(rows[p], acc.at[cd.at[u]], ss[p], add=True)
            if u + 1 < BLK:
                if u >= 1:
                    pltpu.make_async_copy(rows[q], acc.at[cd.at[u]],
                                          ss[q]).wait()
                pltpu.async_copy(xa.at[cs.at[u + 1]], rows[q], sg[q])
        pltpu.make_async_copy(rows[0], acc.at[cd.at[0]], ss[0]).wait()
        pltpu.make_async_copy(rows[1], acc.at[cd.at[0]], ss[1]).wait()

        @pl.when(prefetch)
        def _():
            _wait_block(srcr, ns, nd, si)
            _localize_block(nd, off, dump)

    def pair(i, _):
        b1 = 2 * i + 1
        do_block(b1 - 1, sA, dA, b1 < NBLK, sB, dB)

        @pl.when(b1 < NBLK)
        def _():
            do_block(b1, sB, dB, b1 + 1 < NBLK, sA, dA)

        return 0

    lax.fori_loop(0, (NBLK + 1) // 2, pair, 0)
    plsc.subcore_barrier()
    _copy_out(acc, y_out, s, off)


_conv_call = pl.kernel(
    _conv_body,
    out_type=jax.ShapeDtypeStruct((N_PAD, EMB), jnp.float32),
    mesh=plsc.VectorSubcoreMesh(**_MESH),
    compiler_params=_SC_PARAMS,
    scratch_types=[
        pltpu.VMEM((BLK, CH), jnp.int32),
        pltpu.VMEM((BLK, CH), jnp.int32),
        pltpu.VMEM((BLK, CH), jnp.int32),
        pltpu.VMEM((BLK, CH), jnp.int32),
        pltpu.VMEM((CH, EMB), jnp.float32),
        pltpu.VMEM((CH, EMB), jnp.float32),
        pltpu.SemaphoreType.DMA,
        pltpu.SemaphoreType.DMA,
        pltpu.SemaphoreType.DMA,
        pltpu.SemaphoreType.DMA,
        pltpu.SemaphoreType.DMA,
        pltpu.VMEM_SHARED((H + NDUMP, EMB), jnp.float32),
    ],
)


def _tda_body(xa, tgt, csrc, srcr, dstr, y_out,
              sA, dA, sB, dB, rx0, rx1, rows_t, ca0, ca1,
              sx0, sx1, st, ssc0, ssc1, si, acc):
    c = lax.axis_index("c")
    s = lax.axis_index("s")
    off = c * H
    base = s * RPT
    dump = _dump_vec()
    rx = (rx0, rx1)
    ca = (ca0, ca1)
    sx = (sx0, sx1)
    ssc = (ssc0, ssc1)

    _zero_rows(rx0)
    _zero_acc(acc, rx0, s)

    _stage_block(srcr, dstr, 0, base, sA, dA, si)
    _wait_block(srcr, sA, dA, si)
    plsc.subcore_barrier()
    pltpu.async_copy(xa.at[sA.at[0]], rx0, sx0)
    pltpu.async_copy(csrc.at[sA.at[0]], ca0, sx0)

    def group16(rows_x, catt, eb, _):
        base_e = eb * LANES
        lanes = lax.iota(jnp.int32, LANES)

        def dot1(j, tv):
            e = base_e + j
            xs = [rows_x[e, pl.ds(l * LANES, LANES)]
                  for l in range(EMB // LANES)]
            ts = [rows_t[e, pl.ds(l * LANES, LANES)]
                  for l in range(EMB // LANES)]
            sv = xs[0] * ts[0]
            for l in range(1, EMB // LANES):
                sv = sv + xs[l] * ts[l]
            return jnp.where(lanes == j, jnp.sum(sv), tv)

        tv = lax.fori_loop(0, LANES, dot1, jnp.zeros((LANES,), jnp.float32))
        z = tv * catt[pl.ds(base_e, LANES)]
        att16 = 1.0 / (1.0 + jnp.exp(-z))

        def scale1(j, _):
            e = base_e + j
            av = att16.at[jnp.full((LANES,), j, jnp.int32)].get(
                mode="promise_in_bounds")
            for l in range(EMB // LANES):
                rows_x[e, pl.ds(l * LANES, LANES)] = (
                    rows_x[e, pl.ds(l * LANES, LANES)] * av)
            return 0

        lax.fori_loop(0, LANES, scale1, 0)
        return 0

    def do_chunk(g, cs, cd, nxt_s, u, more):
        p = u % 2
        q = (u + 1) % 2
        dt = pltpu.async_copy(tgt.at[cd.at[u]], rows_t, st)
        pltpu.make_async_copy(xa.at[cs.at[u]], rx[p], sx[p]).wait()
        pltpu.make_async_copy(csrc.at[cs.at[u]], ca[p], sx[p]).wait()

        @pl.when(g > 0)
        def _():
            pltpu.make_async_copy(rx[q], acc.at[cd.at[0]], ssc[q]).wait()

        if u + 1 < BLK:
            pltpu.async_copy(xa.at[cs.at[u + 1]], rx[q], sx[q])
            pltpu.async_copy(csrc.at[cs.at[u + 1]], ca[q], sx[q])
        else:
            @pl.when(more)
            def _():
                pltpu.async_copy(xa.at[nxt_s.at[0]], rx[q], sx[q])
                pltpu.async_copy(csrc.at[nxt_s.at[0]], ca[q], sx[q])
        dt.wait()
        lax.fori_loop(0, CH // LANES,
                      functools.partial(group16, rx[p], ca[p]), 0)
        for l in range(CH // LANES):
            d = cd[u, pl.ds(l * LANES, LANES)]
            t = d - off
            ok = jnp.logical_and(t >= 0, t < H)
            cd[u, pl.ds(l * LANES, LANES)] = jnp.where(ok, t, dump)
        pltpu.async_copy(rx[p], acc.at[cd.at[u]], ssc[p], add=True)

    def do_block(b, cs, cd, prefetch, ns, nd):
        @pl.when(prefetch)
        def _():
            _stage_block(srcr, dstr, b + 1, base, ns, nd, si)

        for u in range(BLK):
            do_chunk(b * BLK + u, cs, cd, ns, u, prefetch)
            if u == 5:
                @pl.when(prefetch)
                def _():
                    _wait_block(srcr, ns, nd, si)

    def pair(i, _):
        b1 = 2 * i + 1
        do_block(b1 - 1, sA, dA, b1 < NBLK, sB, dB)

        @pl.when(b1 < NBLK)
        def _():
            do_block(b1, sB, dB, b1 + 1 < NBLK, sA, dA)

        return 0

    lax.fori_loop(0, (NBLK + 1) // 2, pair, 0)
    pltpu.make_async_copy(rx[(RPT - 1) % 2], acc.at[dB.at[0]],
                          ssc[(RPT - 1) % 2]).wait()
    plsc.subcore_barrier()
    _copy_out(acc, y_out, s, off)


_tda_call = pl.kernel(
    _tda_body,
    out_type=jax.ShapeDtypeStruct((N_PAD, EMB), jnp.float32),
    mesh=plsc.VectorSubcoreMesh(**_MESH),
    compiler_params=_SC_PARAMS,
    scratch_types=[
        pltpu.VMEM((BLK, CH), jnp.int32),
        pltpu.VMEM((BLK, CH), jnp.int32),
        pltpu.VMEM((BLK, CH), jnp.int32),
        pltpu.VMEM((BLK, CH), jnp.int32),
        pltpu.VMEM((CH, EMB), jnp.float32),
        pltpu.VMEM((CH, EMB), jnp.float32),
        pltpu.VMEM((CH, EMB), jnp.float32),
        pltpu.VMEM((CH,), jnp.float32),
        pltpu.VMEM((CH,), jnp.float32),
        pltpu.SemaphoreType.DMA,
        pltpu.SemaphoreType.DMA,
        pltpu.SemaphoreType.DMA,
        pltpu.SemaphoreType.DMA,
        pltpu.SemaphoreType.DMA,
        pltpu.SemaphoreType.DMA,
        pltpu.VMEM_SHARED((H + NDUMP, EMB), jnp.float32),
    ],
)


_TC_R = 1792
_TC_GRID = N_PAD // _TC_R


def _scale_body(x_ref, s_ref, o_ref):
    o_ref[...] = x_ref[...] * s_ref[...]


_scale_call = pl.pallas_call(
    _scale_body,
    grid=(_TC_GRID,),
    in_specs=[pl.BlockSpec((_TC_R, EMB), lambda i: (i, 0)),
              pl.BlockSpec((_TC_R, 1), lambda i: (i, 0))],
    out_specs=pl.BlockSpec((_TC_R, EMB), lambda i: (i, 0)),
    out_shape=jax.ShapeDtypeStruct((N_PAD, EMB), jnp.float32),
)


def _post_body(y_ref, b_ref, s_ref, acc_ref, acco_ref, xa_ref, *, inv_k):
    y = y_ref[...] * b_ref[...]
    n2 = jnp.sum(y * y, axis=1, keepdims=True)
    xn = y * lax.rsqrt(jnp.maximum(n2, 1e-24))
    acco_ref[...] = acc_ref[...] + xn * inv_k
    xa_ref[...] = xn * s_ref[...]


def _make_post(inv_k):
    return pl.pallas_call(
        functools.partial(_post_body, inv_k=inv_k),
        grid=(_TC_GRID,),
        in_specs=[pl.BlockSpec((_TC_R, EMB), lambda i: (i, 0)),
                  pl.BlockSpec((_TC_R, 1), lambda i: (i, 0)),
                  pl.BlockSpec((_TC_R, 1), lambda i: (i, 0)),
                  pl.BlockSpec((_TC_R, EMB), lambda i: (i, 0))],
        out_specs=[pl.BlockSpec((_TC_R, EMB), lambda i: (i, 0)),
                   pl.BlockSpec((_TC_R, EMB), lambda i: (i, 0))],
        out_shape=[jax.ShapeDtypeStruct((N_PAD, EMB), jnp.float32),
                   jax.ShapeDtypeStruct((N_PAD, EMB), jnp.float32)],
    )


_post1 = _make_post(1.0)
_post2 = _make_post(0.5)


def _mga_body(x1, x2, x3, x4, x5, w_ref, b_ref, q_ref, o_ref):
    w = w_ref[...]
    bb = b_ref[...]
    qq = q_ref[...]
    xs = [x1[...], x2[...], x3[...], x4[...], x5[...]]
    sc = []
    for x in xs:
        h = jnp.tanh(jnp.dot(x, w, preferred_element_type=jnp.float32) + bb)
        sc.append(jnp.sum(h * qq, axis=1, keepdims=True))
    m = sc[0]
    for k in range(1, 5):
        m = jnp.maximum(m, sc[k])
    es = [jnp.exp(v - m) for v in sc]
    zsum = es[0]
    for k in range(1, 5):
        zsum = zsum + es[k]
    out = xs[0] * (es[0] / zsum)
    for k in range(1, 5):
        out = out + xs[k] * (es[k] / zsum)
    o_ref[...] = out


_mga_call = pl.pallas_call(
    _mga_body,
    grid=(_TC_GRID,),
    in_specs=[pl.BlockSpec((_TC_R, EMB), lambda i: (i, 0))] * 5 +
             [pl.BlockSpec((EMB, EMB), lambda i: (0, 0)),
              pl.BlockSpec((1, EMB), lambda i: (0, 0)),
              pl.BlockSpec((1, EMB), lambda i: (0, 0))],
    out_specs=pl.BlockSpec((_TC_R, EMB), lambda i: (i, 0)),
    out_shape=jax.ShapeDtypeStruct((N_PAD, EMB), jnp.float32),
)


def _prep_edges(e):
    pad = jnp.full((E_PAD - e.shape[1],), N_NODES, jnp.int32)
    sr = jnp.concatenate([e[0], pad]).reshape(TOT_ROWS, CH)
    dr = jnp.concatenate([e[1], pad]).reshape(TOT_ROWS, CH)
    return sr, dr


def _propagate(x0, sr, dr, a, b):
    ac = a.reshape(N_PAD, 1)
    bc = b.reshape(N_PAD, 1)
    xa = _scale_call(x0, ac)
    y = _conv_call(xa, sr, dr)
    acc, xa = _post1(y, bc, ac, x0)
    y = _conv_call(xa, sr, dr)
    acc, _ = _post2(y, bc, ac, acc)
    return acc


def _propagate_tda(x0, sr, dr, a, b, cinv, tgt):
    ac = a.reshape(N_PAD, 1)
    bc = b.reshape(N_PAD, 1)
    xa = _scale_call(x0, ac)
    y = _tda_call(xa, tgt, cinv, sr, dr)
    acc, xa = _post1(y, bc, ac, x0)
    y = _tda_call(xa, tgt, cinv, sr, dr)
    acc, _ = _post2(y, bc, ac, acc)
    return acc


def kernel(user_emb, item_emb, edge_ubg, edge_view, edge_buy, edge_view_tcb,
           edge_buy_tib, mga_W, mga_b, mga_q):
    x0 = jnp.zeros((N_PAD, EMB), jnp.float32)
    x0 = x0.at[:N_NODES].set(jnp.concatenate([user_emb, item_emb], axis=0))

    sr_u, dr_u = _prep_edges(edge_ubg)
    sr_v, dr_v = _prep_edges(edge_view)
    sr_b, dr_b = _prep_edges(edge_buy)
    sr_vt, dr_vt = _prep_edges(edge_view_tcb)
    sr_bt, dr_bt = _prep_edges(edge_buy_tib)

    a_u, b_u, _ = _deg_call(sr_u, dr_u)
    a_v, b_v, _ = _deg_call(sr_v, dr_v)
    a_b, b_b, _ = _deg_call(sr_b, dr_b)
    a_vt, b_vt, c_vt = _deg_call(sr_vt, dr_vt)
    a_bt, b_bt, _ = _deg_call(sr_bt, dr_bt)

    ubg = _propagate(x0, sr_u, dr_u, a_u, b_u)
    view = _propagate(ubg, sr_v, dr_v, a_v, b_v)
    buy = _propagate(ubg, sr_b, dr_b, a_b, b_b)
    buy_tib = _propagate(buy, sr_bt, dr_bt, a_bt, b_bt)
    view_tcb = _propagate_tda(view, sr_vt, dr_vt, a_vt, b_vt, c_vt, buy)

    final = _mga_call(ubg, view, buy, view_tcb, buy_tib,
                      mga_W, mga_b.reshape(1, EMB), mga_q.reshape(1, EMB))
    return final[:N_NODES]
